# baseline (speedup 1.0000x reference)
import functools

import jax
import jax.numpy as jnp
from jax import lax
from jax.experimental import pallas as pl
from jax.experimental.pallas import tpu as pltpu
from jax.experimental.pallas import tpu_sc as plsc

N = 100000
B = 64
L = 16
NW = 16
CH = N // NW
ALIGN_CH = 6248
WIN = 6280
FULL_IT = (CH - 10) // L
TBL = B * 6 * L
SPAN = TBL // NW
BLK = 2048


def _prep_body(v_ref, oxx, oyy, ozz, oxy, oyz, oxz):
    v = v_ref[...]
    x = v[:, 0]
    y = v[:, 1]
    z = v[:, 2]
    oxx[...] = x * x
    oyy[...] = y * y
    ozz[...] = z * z
    oxy[...] = x * y
    oyz[...] = y * z
    oxz[...] = x * z


_prep = pl.pallas_call(
    _prep_body,
    grid=((N + BLK - 1) // BLK,),
    in_specs=[pl.BlockSpec((BLK, 3), lambda i: (i, 0))],
    out_specs=[pl.BlockSpec((BLK,), lambda i: (i,))] * 6,
    out_shape=[jax.ShapeDtypeStruct((N,), jnp.float32)] * 6,
)


def _sc_body(batch_hbm, xx_hbm, yy_hbm, zz_hbm, xy_hbm, yz_hbm, xz_hbm,
             vol_hbm, pe_hbm, stress_out, energy_out,
             bvec, p0, p1, p2, p3, p4, p5, tbl, red, comb, ctbl, stage,
             vol_v, pe_v, e_v, shared, shared2):
    cid = lax.axis_index("c")
    sid = lax.axis_index("s")
    iota = lax.iota(jnp.int32, L)
    c0 = iota * 0
    cols = [iota + L * c for c in range(6)]
    fzero = jnp.zeros((L,), jnp.float32)
    prods = [p0, p1, p2, p3, p4, p5]

    @pl.when(cid == 0)
    def _phase_a():
        base = sid * ALIGN_CH
        skew = sid * 2
        pltpu.sync_copy(batch_hbm.at[pl.ds(base, WIN)], bvec)
        for ref, hbm in zip(prods, (xx_hbm, yy_hbm, zz_hbm, xy_hbm, yz_hbm,
                                    xz_hbm)):
            pltpu.sync_copy(hbm.at[pl.ds(base, WIN)], ref)

        def _zero(i, c):
            tbl[pl.ds(i * L, L)] = fzero
            return c

        lax.fori_loop(0, TBL // L, _zero, 0)

        def _accum(rows, mask):
            a = plsc.load_gather(bvec, [rows]) * 96
            for c in range(6):
                v = plsc.load_gather(prods[c], [rows])
                plsc.addupdate_scatter(tbl, [a + cols[c]], v, mask=mask)

        def _step(i, c):
            _accum(skew + i * L + iota, None)
            return c

        lax.fori_loop(0, FULL_IT, _step, 0)
        tail = skew + FULL_IT * L + iota
        _accum(jnp.minimum(tail, WIN - 1), iota < (CH - FULL_IT * L))

        pltpu.sync_copy(tbl, shared.at[sid])

    plsc.subcore_barrier()

    @pl.when(cid == 0)
    def _phase_b():
        pltpu.sync_copy(shared.at[:, pl.ds(sid * SPAN, SPAN)], red)
        for k in range(SPAN // L):
            acc = red[0, pl.ds(k * L, L)]
            for j in range(1, NW):
                acc = acc + red[j, pl.ds(k * L, L)]
            comb[pl.ds(k * L, L)] = acc
        pltpu.sync_copy(comb, shared2.at[pl.ds(sid * SPAN, SPAN)])

    plsc.subcore_barrier()

    @pl.when(jnp.logical_and(cid == 0, sid == 0))
    def _phase_c():
        pltpu.sync_copy(shared2, ctbl)
        pltpu.sync_copy(vol_hbm, vol_v)
        for blk in range(4):
            rows = iota + blk * L
            r96 = rows * 96
            vv = vol_v[pl.ds(blk * L, L)]
            for c in range(6):
                acc = fzero
                for lane in range(L):
                    acc = acc + plsc.load_gather(ctbl, [r96 + (c * L + lane)])
                plsc.store_scatter(stage, [rows, c0 + c], acc / vv)
        pltpu.sync_copy(stage, stress_out)

    @pl.when(jnp.logical_and(cid == 0, sid == 1))
    def _phase_e():
        pltpu.sync_copy(pe_hbm, pe_v)
        for blk in range(4):
            ev = plsc.load_gather(pe_v, [iota + blk * L, c0])
            e_v[pl.ds(blk * L, L)] = ev
        pltpu.sync_copy(e_v, energy_out)


_sc_all = functools.partial(
    pl.kernel,
    out_type=(
        jax.ShapeDtypeStruct((B, 6), jnp.float32),
        jax.ShapeDtypeStruct((B,), jnp.float32),
    ),
    mesh=plsc.VectorSubcoreMesh(
        core_axis_name="c", subcore_axis_name="s", num_cores=2, num_subcores=16
    ),
    scratch_types=[
        pltpu.VMEM((WIN,), jnp.int32),
        pltpu.VMEM((WIN,), jnp.float32),
        pltpu.VMEM((WIN,), jnp.float32),
        pltpu.VMEM((WIN,), jnp.float32),
        pltpu.VMEM((WIN,), jnp.float32),
        pltpu.VMEM((WIN,), jnp.float32),
        pltpu.VMEM((WIN,), jnp.float32),
        pltpu.VMEM((TBL,), jnp.float32),
        pltpu.VMEM((NW, SPAN), jnp.float32),
        pltpu.VMEM((SPAN,), jnp.float32),
        pltpu.VMEM((TBL,), jnp.float32),
        pltpu.VMEM((B, 6), jnp.float32),
        pltpu.VMEM((B,), jnp.float32),
        pltpu.VMEM((B, 1), jnp.float32),
        pltpu.VMEM((B,), jnp.float32),
        pltpu.VMEM_SHARED((NW, TBL), jnp.float32),
        pltpu.VMEM_SHARED((TBL,), jnp.float32),
    ],
    compiler_params=pltpu.CompilerParams(
        needs_layout_passes=False, use_tc_tiling_on_sc=False
    ),
)(_sc_body)


def kernel(pred_energy, pred_force, atomic_stress, cell_volume, batch):
    del pred_force
    xx, yy, zz, xy, yz, xz = _prep(atomic_stress)
    stress, energy = _sc_all(
        batch.astype(jnp.int32), xx, yy, zz, xy, yz, xz, cell_volume,
        pred_energy
    )
    return (energy, stress)
